# baseline (speedup 1.0000x reference)
import functools

import jax
import jax.numpy as jnp
from jax import lax
from jax.experimental import pallas as pl
from jax.experimental.pallas import tpu as pltpu
from jax.experimental.pallas import tpu_sc as plsc

N = 10000
E = 320000
C = 128
NC = 2
NS = 16
NW = NC * NS
NP = 10240
ROWS_PER_TILE = NP // NS
CH = 64
NCHUNK = E // CH
HW = 16
EPW = E // NW
BN = 2000
NK = 160
EP = NK * NW * CH
NSLOT = 4


def _sne_body(x_ref, ntc_ref, ntt_ref, o_ref):
    iota16 = lax.broadcasted_iota(jnp.int32, (1, 16), 1)
    onehot = (ntc_ref[...] == iota16).astype(jnp.float32)
    o_ref[...] = x_ref[...] + jnp.dot(onehot, ntt_ref[...],
                                      preferred_element_type=jnp.float32)


_sne_call = pl.pallas_call(
    _sne_body,
    grid=(N // BN,),
    in_specs=[
        pl.BlockSpec((BN, C), lambda i: (i, 0)),
        pl.BlockSpec((BN, 1), lambda i: (i, 0)),
        pl.BlockSpec((16, C), lambda i: (0, 0)),
    ],
    out_specs=pl.BlockSpec((BN, C), lambda i: (i, 0)),
    out_shape=jax.ShapeDtypeStruct((N, C), jnp.float32),
)


def _seg_body(sne_hbm, src_hbm, dst_hbm, et_hbm, oht_hbm, zrow_hbm, zhist_hbm,
              s_out, hist_out,
              acc_sh, hist_sh, sidx_v, didx_v, etx_v, rows_v, oh_v,
              sem_i0, sem_i1, sem_g0, sem_g1, sem_s0, sem_s1):
    c_id = lax.axis_index("c")
    s_id = lax.axis_index("s")
    wid = s_id * NC + c_id

    base = s_id * ROWS_PER_TILE
    pltpu.sync_copy(zrow_hbm, rows_v.at[0])
    for b in range(NSLOT):
        pltpu.sync_copy(zhist_hbm, oh_v.at[b])
    for j in range(ROWS_PER_TILE // CH):
        rb = base + j * CH
        pltpu.sync_copy(rows_v.at[0], acc_sh.at[pl.ds(rb, CH)])
        pltpu.sync_copy(oh_v.at[0], hist_sh.at[pl.ds(rb, CH)])

    plsc.subcore_barrier()

    sem_i = (sem_i0, sem_i1, sem_i0, sem_i1)
    sem_g = (sem_g0, sem_g1, sem_g0, sem_g1)
    sem_s = (sem_s0, sem_s1, sem_s0, sem_s1)
    cpy = pltpu.async_copy

    def chunk_body(g, carry):
        di, dg, ds_ = [], [], []
        for b in range(NSLOT):
            off = (wid + (NSLOT * g + b) * NW) * CH
            di.append((
                cpy(src_hbm.at[pl.ds(off, CH)], sidx_v.at[b], sem_i[b]),
                cpy(dst_hbm.at[pl.ds(off, CH)], didx_v.at[b], sem_i[b]),
                cpy(et_hbm.at[pl.ds(off, CH)], etx_v.at[b], sem_i[b]),
            ))
        for b in range(NSLOT):
            for d in di[b]:
                d.wait()
        for b in range(NSLOT):
            dg.append((
                cpy(sne_hbm.at[sidx_v.at[b]], rows_v.at[b], sem_g[b]),
            ))
        ones = jnp.ones((16,), jnp.float32)
        z16 = jnp.zeros((16,), jnp.float32)
        for b in range(NSLOT):
            for gg in range(CH // 16):
                ev = lax.iota(jnp.int32, 16) + (gg * 16)
                etv = etx_v[b, pl.ds(gg * 16, 16)]
                plsc.store_scatter(oh_v.at[b], [ev, etv], ones)
        for b in range(NSLOT):
            for d in dg[b]:
                d.wait()
        for b in range(NSLOT):
            ds_.append((
                cpy(rows_v.at[b], acc_sh.at[didx_v.at[b]], sem_s[b],
                    add=True),
                cpy(oh_v.at[b], hist_sh.at[didx_v.at[b]], sem_s[b],
                    add=True),
            ))
        for b in range(NSLOT):
            for d in ds_[b]:
                d.wait()
        for b in range(NSLOT):
            for gg in range(CH // 16):
                ev = lax.iota(jnp.int32, 16) + (gg * 16)
                etv = etx_v[b, pl.ds(gg * 16, 16)]
                plsc.store_scatter(oh_v.at[b], [ev, etv], z16)
        return carry

    lax.fori_loop(0, NK // NSLOT, chunk_body, 0)

    plsc.subcore_barrier()

    for j in range(ROWS_PER_TILE // CH):
        rb = base + j * CH
        pltpu.sync_copy(acc_sh.at[pl.ds(rb, CH)], rows_v.at[0])
        pltpu.sync_copy(rows_v.at[0], s_out.at[c_id, pl.ds(rb, CH)])
        pltpu.sync_copy(hist_sh.at[pl.ds(rb, CH)], oh_v.at[0])
        pltpu.sync_copy(oh_v.at[0], hist_out.at[c_id, pl.ds(rb, CH)])


_seg_call = pl.kernel(
    _seg_body,
    out_type=[
        jax.ShapeDtypeStruct((NC, NP, C), jnp.float32),
        jax.ShapeDtypeStruct((NC, NP, HW), jnp.float32),
    ],
    mesh=plsc.VectorSubcoreMesh(core_axis_name="c", subcore_axis_name="s"),
    scratch_types=[
        pltpu.VMEM_SHARED((NP, C), jnp.float32),
        pltpu.VMEM_SHARED((NP, HW), jnp.float32),
        pltpu.VMEM((NSLOT, CH), jnp.int32),
        pltpu.VMEM((NSLOT, CH), jnp.int32),
        pltpu.VMEM((NSLOT, CH), jnp.int32),
        pltpu.VMEM((NSLOT, CH, C), jnp.float32),
        pltpu.VMEM((NSLOT, CH, HW), jnp.float32),
        pltpu.SemaphoreType.DMA,
        pltpu.SemaphoreType.DMA,
        pltpu.SemaphoreType.DMA,
        pltpu.SemaphoreType.DMA,
        pltpu.SemaphoreType.DMA,
        pltpu.SemaphoreType.DMA,
    ],
    compiler_params=pltpu.CompilerParams(needs_layout_passes=False, use_tc_tiling_on_sc=False),
)


def _combine_body(sne_ref, s_ref, h_ref, ett_ref,
                  w1_ref, w2_ref, wrt_ref, wnb_ref, wrb_ref, a12_ref,
                  out_a1_ref, out_a2_ref, out_ge_ref):
    S = s_ref[0] + s_ref[1]
    ethist = h_ref[0, :, :8] + h_ref[1, :, :8]
    cnt = jnp.sum(ethist, axis=1, keepdims=True)
    denom = jnp.maximum(cnt, 1.0)
    efm = (S - jnp.dot(ethist, ett_ref[...],
                       preferred_element_type=jnp.float32)) / denom
    sne = sne_ref[...]
    wc = w2_ref[...] + wrt_ref[...]
    h = (jnp.dot(sne, w1_ref[...], preferred_element_type=jnp.float32)
         + jnp.dot(efm, wc, preferred_element_type=jnp.float32)
         + wnb_ref[...]
         + jnp.where(cnt > 0.0, 1.0, 0.0) * wrb_ref[...])
    a12 = jnp.dot(h, a12_ref[...], preferred_element_type=jnp.float32)
    out_a1_ref[...] = a12[:, 0:1]
    out_a2_ref[...] = a12[:, 1:2]

    @pl.when(pl.program_id(0) == 0)
    def _():
        out_ge_ref[...] = jnp.zeros_like(out_ge_ref)

    out_ge_ref[...] += jnp.sum(h, axis=0, keepdims=True) * (1.0 / N)


_combine_call = pl.pallas_call(
    _combine_body,
    grid=(N // BN,),
    in_specs=[
        pl.BlockSpec((BN, C), lambda i: (i, 0)),
        pl.BlockSpec((NC, BN, C), lambda i: (0, i, 0)),
        pl.BlockSpec((NC, BN, HW), lambda i: (0, i, 0)),
        pl.BlockSpec((8, C), lambda i: (0, 0)),
        pl.BlockSpec((C, C), lambda i: (0, 0)),
        pl.BlockSpec((C, C), lambda i: (0, 0)),
        pl.BlockSpec((C, C), lambda i: (0, 0)),
        pl.BlockSpec((1, C), lambda i: (0, 0)),
        pl.BlockSpec((1, C), lambda i: (0, 0)),
        pl.BlockSpec((C, 2), lambda i: (0, 0)),
    ],
    out_specs=[
        pl.BlockSpec((BN, 1), lambda i: (i, 0)),
        pl.BlockSpec((BN, 1), lambda i: (i, 0)),
        pl.BlockSpec((1, C), lambda i: (0, 0)),
    ],
    out_shape=[
        jax.ShapeDtypeStruct((N, 1), jnp.float32),
        jax.ShapeDtypeStruct((N, 1), jnp.float32),
        jax.ShapeDtypeStruct((1, C), jnp.float32),
    ],
)


def _logits_body(a1_hbm, a2_hbm, src_hbm, dst_hbm, out_hbm,
                 a1_v, a2_v, sidx_v, didx_v, lg_v):
    c_id = lax.axis_index("c")
    s_id = lax.axis_index("s")
    wid = s_id * NC + c_id
    eoff = wid * EPW
    pltpu.sync_copy(a1_hbm, a1_v)
    pltpu.sync_copy(a2_hbm, a2_v)
    pltpu.sync_copy(src_hbm.at[pl.ds(eoff, EPW)], sidx_v)
    pltpu.sync_copy(dst_hbm.at[pl.ds(eoff, EPW)], didx_v)

    def body(g, carry):
        o = pl.multiple_of(g * 16, 16)
        sv = sidx_v[pl.ds(o, 16)]
        dv = didx_v[pl.ds(o, 16)]
        lg_v[pl.ds(o, 16)] = (plsc.load_gather(a1_v, [sv])
                              + plsc.load_gather(a2_v, [dv]))
        return carry

    lax.fori_loop(0, EPW // 16, body, 0)
    pltpu.sync_copy(lg_v, out_hbm.at[pl.ds(eoff, EPW)])


_logits_call = pl.kernel(
    _logits_body,
    out_type=jax.ShapeDtypeStruct((E,), jnp.float32),
    mesh=plsc.VectorSubcoreMesh(core_axis_name="c", subcore_axis_name="s"),
    scratch_types=[
        pltpu.VMEM((N,), jnp.float32),
        pltpu.VMEM((N,), jnp.float32),
        pltpu.VMEM((EPW,), jnp.int32),
        pltpu.VMEM((EPW,), jnp.int32),
        pltpu.VMEM((EPW,), jnp.float32),
    ],
    compiler_params=pltpu.CompilerParams(needs_layout_passes=False, use_tc_tiling_on_sc=False),
)


def _softmax_body(l_ref, o_ref):
    l = l_ref[...]
    m = jnp.max(l)
    e = jnp.exp(l - m)
    o_ref[...] = e / jnp.sum(e)


_softmax_call = pl.pallas_call(
    _softmax_body,
    out_shape=jax.ShapeDtypeStruct((E // C, C), jnp.float32),
)


def kernel(x, edge_index, node_type, edge_type, node_type_table,
           edge_type_table, WN_w, WN_b, WR_w, WR_b, A_w, A_b):
    src = edge_index[0].astype(jnp.int32)
    dst = edge_index[1].astype(jnp.int32)
    nt_i = node_type.astype(jnp.int32)
    et_i = edge_type.astype(jnp.int32)

    sne = _sne_call(x, nt_i.reshape(N, 1), node_type_table)

    padi = jnp.arange(EP - E, dtype=jnp.int32)
    srcp = jnp.concatenate([src, padi % N])
    dstp = jnp.concatenate([dst, N + padi % (NP - N)])
    etp = jnp.concatenate([et_i, padi % 8])
    oht = jnp.eye(8, HW, dtype=jnp.float32)
    zrow = jnp.zeros((CH, C), jnp.float32)
    zhist = jnp.zeros((CH, HW), jnp.float32)
    s_part, hist_part = _seg_call(sne, srcp, dstp, etp, oht, zrow, zhist)

    w1 = WN_w[:, :C].T
    w2 = WN_w[:, C:].T
    wrt = WR_w.T
    a12 = A_w[0].reshape(2, C).T
    a1_out, a2_out, ge = _combine_call(sne, s_part, hist_part, edge_type_table,
                                       w1, w2, wrt,
                                       WN_b.reshape(1, C), WR_b.reshape(1, C),
                                       a12)

    logits = _logits_call(a1_out.reshape(N), a2_out.reshape(N), src, dst)
    aw = _softmax_call(logits.reshape(E // C, C)).reshape(E)
    return ge.reshape(C), aw


# speedup vs baseline: 1.0084x; 1.0084x over previous
import functools

import jax
import jax.numpy as jnp
from jax import lax
from jax.experimental import pallas as pl
from jax.experimental.pallas import tpu as pltpu
from jax.experimental.pallas import tpu_sc as plsc

N = 10000
E = 320000
C = 128
NC = 2
NS = 16
NW = NC * NS
NP = 10240
ROWS_PER_TILE = NP // NS
CH = 128
NCHUNK = E // CH
HW = 16
EPW = E // NW
BN = 2000
NK = 80
EP = NK * NW * CH
NSLOT = 2


def _sne_body(x_ref, ntc_ref, ntt_ref, o_ref):
    iota16 = lax.broadcasted_iota(jnp.int32, (1, 16), 1)
    onehot = (ntc_ref[...] == iota16).astype(jnp.float32)
    o_ref[...] = x_ref[...] + jnp.dot(onehot, ntt_ref[...],
                                      preferred_element_type=jnp.float32)


_sne_call = pl.pallas_call(
    _sne_body,
    grid=(N // BN,),
    in_specs=[
        pl.BlockSpec((BN, C), lambda i: (i, 0)),
        pl.BlockSpec((BN, 1), lambda i: (i, 0)),
        pl.BlockSpec((16, C), lambda i: (0, 0)),
    ],
    out_specs=pl.BlockSpec((BN, C), lambda i: (i, 0)),
    out_shape=jax.ShapeDtypeStruct((N, C), jnp.float32),
)


def _seg_body(sne_hbm, src_hbm, dst_hbm, et_hbm, oht_hbm, zrow_hbm, zhist_hbm,
              s_out, hist_out,
              acc_sh, hist_sh, sidx_v, didx_v, etx_v, rows_v, oh_v,
              sem_i0, sem_i1, sem_g0, sem_g1, sem_s0, sem_s1):
    c_id = lax.axis_index("c")
    s_id = lax.axis_index("s")
    wid = s_id * NC + c_id

    base = s_id * ROWS_PER_TILE
    pltpu.sync_copy(zrow_hbm, rows_v.at[0])
    for b in range(NSLOT):
        pltpu.sync_copy(zhist_hbm, oh_v.at[b])
    for j in range(ROWS_PER_TILE // CH):
        rb = base + j * CH
        pltpu.sync_copy(rows_v.at[0], acc_sh.at[pl.ds(rb, CH)])
        pltpu.sync_copy(oh_v.at[0], hist_sh.at[pl.ds(rb, CH)])

    plsc.subcore_barrier()

    sem_i = (sem_i0, sem_i1)
    sem_g = (sem_g0, sem_g1)
    sem_s = (sem_s0, sem_s1)
    cpy = pltpu.async_copy

    def chunk_body(g, carry):
        di, dg, ds_ = [], [], []
        for b in range(NSLOT):
            off = (wid + (NSLOT * g + b) * NW) * CH
            di.append((
                cpy(src_hbm.at[pl.ds(off, CH)], sidx_v.at[b], sem_i[b]),
                cpy(dst_hbm.at[pl.ds(off, CH)], didx_v.at[b], sem_i[b]),
                cpy(et_hbm.at[pl.ds(off, CH)], etx_v.at[b], sem_i[b]),
            ))
        for b in range(NSLOT):
            for d in di[b]:
                d.wait()
        for b in range(NSLOT):
            dg.append((
                cpy(sne_hbm.at[sidx_v.at[b]], rows_v.at[b], sem_g[b]),
            ))
        ones = jnp.ones((16,), jnp.float32)
        z16 = jnp.zeros((16,), jnp.float32)
        for b in range(NSLOT):
            for gg in range(CH // 16):
                ev = lax.iota(jnp.int32, 16) + (gg * 16)
                etv = etx_v[b, pl.ds(gg * 16, 16)]
                plsc.store_scatter(oh_v.at[b], [ev, etv], ones)
        for b in range(NSLOT):
            for d in dg[b]:
                d.wait()
        for b in range(NSLOT):
            ds_.append((
                cpy(rows_v.at[b], acc_sh.at[didx_v.at[b]], sem_s[b],
                    add=True),
                cpy(oh_v.at[b], hist_sh.at[didx_v.at[b]], sem_s[b],
                    add=True),
            ))
        for b in range(NSLOT):
            for d in ds_[b]:
                d.wait()
        for b in range(NSLOT):
            for gg in range(CH // 16):
                ev = lax.iota(jnp.int32, 16) + (gg * 16)
                etv = etx_v[b, pl.ds(gg * 16, 16)]
                plsc.store_scatter(oh_v.at[b], [ev, etv], z16)
        return carry

    lax.fori_loop(0, NK // NSLOT, chunk_body, 0)

    plsc.subcore_barrier()

    for j in range(ROWS_PER_TILE // CH):
        rb = base + j * CH
        pltpu.sync_copy(acc_sh.at[pl.ds(rb, CH)], rows_v.at[0])
        pltpu.sync_copy(rows_v.at[0], s_out.at[c_id, pl.ds(rb, CH)])
        pltpu.sync_copy(hist_sh.at[pl.ds(rb, CH)], oh_v.at[0])
        pltpu.sync_copy(oh_v.at[0], hist_out.at[c_id, pl.ds(rb, CH)])


_seg_call = pl.kernel(
    _seg_body,
    out_type=[
        jax.ShapeDtypeStruct((NC, NP, C), jnp.float32),
        jax.ShapeDtypeStruct((NC, NP, HW), jnp.float32),
    ],
    mesh=plsc.VectorSubcoreMesh(core_axis_name="c", subcore_axis_name="s"),
    scratch_types=[
        pltpu.VMEM_SHARED((NP, C), jnp.float32),
        pltpu.VMEM_SHARED((NP, HW), jnp.float32),
        pltpu.VMEM((NSLOT, CH), jnp.int32),
        pltpu.VMEM((NSLOT, CH), jnp.int32),
        pltpu.VMEM((NSLOT, CH), jnp.int32),
        pltpu.VMEM((NSLOT, CH, C), jnp.float32),
        pltpu.VMEM((NSLOT, CH, HW), jnp.float32),
        pltpu.SemaphoreType.DMA,
        pltpu.SemaphoreType.DMA,
        pltpu.SemaphoreType.DMA,
        pltpu.SemaphoreType.DMA,
        pltpu.SemaphoreType.DMA,
        pltpu.SemaphoreType.DMA,
    ],
    compiler_params=pltpu.CompilerParams(needs_layout_passes=False, use_tc_tiling_on_sc=False),
)


def _combine_body(sne_ref, s_ref, h_ref, ett_ref,
                  w1_ref, w2_ref, wrt_ref, wnb_ref, wrb_ref, a12_ref,
                  out_a1_ref, out_a2_ref, out_ge_ref):
    S = s_ref[0] + s_ref[1]
    ethist = h_ref[0, :, :8] + h_ref[1, :, :8]
    cnt = jnp.sum(ethist, axis=1, keepdims=True)
    denom = jnp.maximum(cnt, 1.0)
    efm = (S - jnp.dot(ethist, ett_ref[...],
                       preferred_element_type=jnp.float32)) / denom
    sne = sne_ref[...]
    wc = w2_ref[...] + wrt_ref[...]
    h = (jnp.dot(sne, w1_ref[...], preferred_element_type=jnp.float32)
         + jnp.dot(efm, wc, preferred_element_type=jnp.float32)
         + wnb_ref[...]
         + jnp.where(cnt > 0.0, 1.0, 0.0) * wrb_ref[...])
    a12 = jnp.dot(h, a12_ref[...], preferred_element_type=jnp.float32)
    out_a1_ref[...] = a12[:, 0:1]
    out_a2_ref[...] = a12[:, 1:2]

    @pl.when(pl.program_id(0) == 0)
    def _():
        out_ge_ref[...] = jnp.zeros_like(out_ge_ref)

    out_ge_ref[...] += jnp.sum(h, axis=0, keepdims=True) * (1.0 / N)


_combine_call = pl.pallas_call(
    _combine_body,
    grid=(N // BN,),
    in_specs=[
        pl.BlockSpec((BN, C), lambda i: (i, 0)),
        pl.BlockSpec((NC, BN, C), lambda i: (0, i, 0)),
        pl.BlockSpec((NC, BN, HW), lambda i: (0, i, 0)),
        pl.BlockSpec((8, C), lambda i: (0, 0)),
        pl.BlockSpec((C, C), lambda i: (0, 0)),
        pl.BlockSpec((C, C), lambda i: (0, 0)),
        pl.BlockSpec((C, C), lambda i: (0, 0)),
        pl.BlockSpec((1, C), lambda i: (0, 0)),
        pl.BlockSpec((1, C), lambda i: (0, 0)),
        pl.BlockSpec((C, 2), lambda i: (0, 0)),
    ],
    out_specs=[
        pl.BlockSpec((BN, 1), lambda i: (i, 0)),
        pl.BlockSpec((BN, 1), lambda i: (i, 0)),
        pl.BlockSpec((1, C), lambda i: (0, 0)),
    ],
    out_shape=[
        jax.ShapeDtypeStruct((N, 1), jnp.float32),
        jax.ShapeDtypeStruct((N, 1), jnp.float32),
        jax.ShapeDtypeStruct((1, C), jnp.float32),
    ],
)


def _logits_body(a1_hbm, a2_hbm, src_hbm, dst_hbm, out_hbm,
                 a1_v, a2_v, sidx_v, didx_v, lg_v):
    c_id = lax.axis_index("c")
    s_id = lax.axis_index("s")
    wid = s_id * NC + c_id
    eoff = wid * EPW
    pltpu.sync_copy(a1_hbm, a1_v)
    pltpu.sync_copy(a2_hbm, a2_v)
    pltpu.sync_copy(src_hbm.at[pl.ds(eoff, EPW)], sidx_v)
    pltpu.sync_copy(dst_hbm.at[pl.ds(eoff, EPW)], didx_v)

    def body(g, carry):
        o = pl.multiple_of(g * 16, 16)
        sv = sidx_v[pl.ds(o, 16)]
        dv = didx_v[pl.ds(o, 16)]
        lg_v[pl.ds(o, 16)] = (plsc.load_gather(a1_v, [sv])
                              + plsc.load_gather(a2_v, [dv]))
        return carry

    lax.fori_loop(0, EPW // 16, body, 0)
    pltpu.sync_copy(lg_v, out_hbm.at[pl.ds(eoff, EPW)])


_logits_call = pl.kernel(
    _logits_body,
    out_type=jax.ShapeDtypeStruct((E,), jnp.float32),
    mesh=plsc.VectorSubcoreMesh(core_axis_name="c", subcore_axis_name="s"),
    scratch_types=[
        pltpu.VMEM((N,), jnp.float32),
        pltpu.VMEM((N,), jnp.float32),
        pltpu.VMEM((EPW,), jnp.int32),
        pltpu.VMEM((EPW,), jnp.int32),
        pltpu.VMEM((EPW,), jnp.float32),
    ],
    compiler_params=pltpu.CompilerParams(needs_layout_passes=False, use_tc_tiling_on_sc=False),
)


def _softmax_body(l_ref, o_ref):
    l = l_ref[...]
    m = jnp.max(l)
    e = jnp.exp(l - m)
    o_ref[...] = e / jnp.sum(e)


_softmax_call = pl.pallas_call(
    _softmax_body,
    out_shape=jax.ShapeDtypeStruct((E // C, C), jnp.float32),
)


def kernel(x, edge_index, node_type, edge_type, node_type_table,
           edge_type_table, WN_w, WN_b, WR_w, WR_b, A_w, A_b):
    src = edge_index[0].astype(jnp.int32)
    dst = edge_index[1].astype(jnp.int32)
    nt_i = node_type.astype(jnp.int32)
    et_i = edge_type.astype(jnp.int32)

    sne = _sne_call(x, nt_i.reshape(N, 1), node_type_table)

    padi = jnp.arange(EP - E, dtype=jnp.int32)
    srcp = jnp.concatenate([src, padi % N])
    dstp = jnp.concatenate([dst, N + padi % (NP - N)])
    etp = jnp.concatenate([et_i, padi % 8])
    oht = jnp.eye(8, HW, dtype=jnp.float32)
    zrow = jnp.zeros((CH, C), jnp.float32)
    zhist = jnp.zeros((CH, HW), jnp.float32)
    s_part, hist_part = _seg_call(sne, srcp, dstp, etp, oht, zrow, zhist)

    w1 = WN_w[:, :C].T
    w2 = WN_w[:, C:].T
    wrt = WR_w.T
    a12 = A_w[0].reshape(2, C).T
    a1_out, a2_out, ge = _combine_call(sne, s_part, hist_part, edge_type_table,
                                       w1, w2, wrt,
                                       WN_b.reshape(1, C), WR_b.reshape(1, C),
                                       a12)

    logits = _logits_call(a1_out.reshape(N), a2_out.reshape(N), src, dst)
    aw = _softmax_call(logits.reshape(E // C, C)).reshape(E)
    return ge.reshape(C), aw


# speedup vs baseline: 1.0225x; 1.0141x over previous
import functools

import jax
import jax.numpy as jnp
from jax import lax
from jax.experimental import pallas as pl
from jax.experimental.pallas import tpu as pltpu
from jax.experimental.pallas import tpu_sc as plsc

N = 10000
E = 320000
C = 128
NC = 2
NS = 16
NW = NC * NS
NP = 10240
ROWS_PER_TILE = NP // NS
CH = 128
NCHUNK = E // CH
HW = 16
EPW = E // NW
BN = 2000
NK = 80
EP = NK * NW * CH
NSLOT = 2


def _sne_body(x_ref, ntc_ref, ntt_ref, o_ref):
    iota16 = lax.broadcasted_iota(jnp.int32, (1, 16), 1)
    onehot = (ntc_ref[...] == iota16).astype(jnp.float32)
    o_ref[...] = x_ref[...] + jnp.dot(onehot, ntt_ref[...],
                                      preferred_element_type=jnp.float32)


_sne_call = pl.pallas_call(
    _sne_body,
    grid=(N // BN,),
    in_specs=[
        pl.BlockSpec((BN, C), lambda i: (i, 0)),
        pl.BlockSpec((BN, 1), lambda i: (i, 0)),
        pl.BlockSpec((16, C), lambda i: (0, 0)),
    ],
    out_specs=pl.BlockSpec((BN, C), lambda i: (i, 0)),
    out_shape=jax.ShapeDtypeStruct((N, C), jnp.float32),
)


def _seg_body(sne_hbm, src_hbm, dst_hbm, et_hbm, oht_hbm, zrow_hbm, zhist_hbm,
              s_out, hist_out,
              acc_sh, hist_sh, sidx_v, didx_v, etx_v, rows_v, oh_v,
              sem_i0, sem_i1, sem_g0, sem_g1, sem_s0, sem_s1):
    c_id = lax.axis_index("c")
    s_id = lax.axis_index("s")
    wid = s_id * NC + c_id

    base = s_id * ROWS_PER_TILE
    pltpu.sync_copy(zrow_hbm, rows_v.at[0])
    for b in range(NSLOT):
        pltpu.sync_copy(zhist_hbm, oh_v.at[b])
    for j in range(ROWS_PER_TILE // CH):
        rb = base + j * CH
        pltpu.sync_copy(rows_v.at[0], acc_sh.at[pl.ds(rb, CH)])
        pltpu.sync_copy(oh_v.at[0], hist_sh.at[pl.ds(rb, CH)])

    plsc.subcore_barrier()

    sem_i = (sem_i0, sem_i1)
    sem_g = (sem_g0, sem_g1)
    sem_s = (sem_s0, sem_s1)
    cpy = pltpu.async_copy

    def chunk_body(g, carry):
        di, dg, ds_ = [], [], []
        for b in range(NSLOT):
            off = (wid + (NSLOT * g + b) * NW) * CH
            di.append((
                cpy(src_hbm.at[pl.ds(off, CH)], sidx_v.at[b], sem_i[b]),
                cpy(dst_hbm.at[pl.ds(off, CH)], didx_v.at[b], sem_i[b]),
                cpy(et_hbm.at[pl.ds(off, CH)], etx_v.at[b], sem_i[b]),
            ))
        for b in range(NSLOT):
            for d in di[b]:
                d.wait()
            dg.append((
                cpy(sne_hbm.at[sidx_v.at[b]], rows_v.at[b], sem_g[b]),
            ))
        ones = jnp.ones((16,), jnp.float32)
        z16 = jnp.zeros((16,), jnp.float32)
        for b in range(NSLOT):
            for gg in range(CH // 16):
                ev = lax.iota(jnp.int32, 16) + (gg * 16)
                etv = etx_v[b, pl.ds(gg * 16, 16)]
                plsc.store_scatter(oh_v.at[b], [ev, etv], ones)
        for b in range(NSLOT):
            for d in dg[b]:
                d.wait()
            ds_.append((
                cpy(rows_v.at[b], acc_sh.at[didx_v.at[b]], sem_s[b],
                    add=True),
                cpy(oh_v.at[b], hist_sh.at[didx_v.at[b]], sem_s[b],
                    add=True),
            ))
        for b in range(NSLOT):
            for d in ds_[b]:
                d.wait()
        for b in range(NSLOT):
            for gg in range(CH // 16):
                ev = lax.iota(jnp.int32, 16) + (gg * 16)
                etv = etx_v[b, pl.ds(gg * 16, 16)]
                plsc.store_scatter(oh_v.at[b], [ev, etv], z16)
        return carry

    lax.fori_loop(0, NK // NSLOT, chunk_body, 0)

    plsc.subcore_barrier()

    for j in range(ROWS_PER_TILE // CH):
        rb = base + j * CH
        pltpu.sync_copy(acc_sh.at[pl.ds(rb, CH)], rows_v.at[0])
        pltpu.sync_copy(rows_v.at[0], s_out.at[c_id, pl.ds(rb, CH)])
        pltpu.sync_copy(hist_sh.at[pl.ds(rb, CH)], oh_v.at[0])
        pltpu.sync_copy(oh_v.at[0], hist_out.at[c_id, pl.ds(rb, CH)])


_seg_call = pl.kernel(
    _seg_body,
    out_type=[
        jax.ShapeDtypeStruct((NC, NP, C), jnp.float32),
        jax.ShapeDtypeStruct((NC, NP, HW), jnp.float32),
    ],
    mesh=plsc.VectorSubcoreMesh(core_axis_name="c", subcore_axis_name="s"),
    scratch_types=[
        pltpu.VMEM_SHARED((NP, C), jnp.float32),
        pltpu.VMEM_SHARED((NP, HW), jnp.float32),
        pltpu.VMEM((NSLOT, CH), jnp.int32),
        pltpu.VMEM((NSLOT, CH), jnp.int32),
        pltpu.VMEM((NSLOT, CH), jnp.int32),
        pltpu.VMEM((NSLOT, CH, C), jnp.float32),
        pltpu.VMEM((NSLOT, CH, HW), jnp.float32),
        pltpu.SemaphoreType.DMA,
        pltpu.SemaphoreType.DMA,
        pltpu.SemaphoreType.DMA,
        pltpu.SemaphoreType.DMA,
        pltpu.SemaphoreType.DMA,
        pltpu.SemaphoreType.DMA,
    ],
    compiler_params=pltpu.CompilerParams(needs_layout_passes=False, use_tc_tiling_on_sc=False),
)


def _combine_body(sne_ref, s_ref, h_ref, ett_ref,
                  w1_ref, w2_ref, wrt_ref, wnb_ref, wrb_ref, a12_ref,
                  out_a1_ref, out_a2_ref, out_ge_ref):
    S = s_ref[0] + s_ref[1]
    ethist = h_ref[0, :, :8] + h_ref[1, :, :8]
    cnt = jnp.sum(ethist, axis=1, keepdims=True)
    denom = jnp.maximum(cnt, 1.0)
    efm = (S - jnp.dot(ethist, ett_ref[...],
                       preferred_element_type=jnp.float32)) / denom
    sne = sne_ref[...]
    wc = w2_ref[...] + wrt_ref[...]
    h = (jnp.dot(sne, w1_ref[...], preferred_element_type=jnp.float32)
         + jnp.dot(efm, wc, preferred_element_type=jnp.float32)
         + wnb_ref[...]
         + jnp.where(cnt > 0.0, 1.0, 0.0) * wrb_ref[...])
    a12 = jnp.dot(h, a12_ref[...], preferred_element_type=jnp.float32)
    out_a1_ref[...] = a12[:, 0:1]
    out_a2_ref[...] = a12[:, 1:2]

    @pl.when(pl.program_id(0) == 0)
    def _():
        out_ge_ref[...] = jnp.zeros_like(out_ge_ref)

    out_ge_ref[...] += jnp.sum(h, axis=0, keepdims=True) * (1.0 / N)


_combine_call = pl.pallas_call(
    _combine_body,
    grid=(N // BN,),
    in_specs=[
        pl.BlockSpec((BN, C), lambda i: (i, 0)),
        pl.BlockSpec((NC, BN, C), lambda i: (0, i, 0)),
        pl.BlockSpec((NC, BN, HW), lambda i: (0, i, 0)),
        pl.BlockSpec((8, C), lambda i: (0, 0)),
        pl.BlockSpec((C, C), lambda i: (0, 0)),
        pl.BlockSpec((C, C), lambda i: (0, 0)),
        pl.BlockSpec((C, C), lambda i: (0, 0)),
        pl.BlockSpec((1, C), lambda i: (0, 0)),
        pl.BlockSpec((1, C), lambda i: (0, 0)),
        pl.BlockSpec((C, 2), lambda i: (0, 0)),
    ],
    out_specs=[
        pl.BlockSpec((BN, 1), lambda i: (i, 0)),
        pl.BlockSpec((BN, 1), lambda i: (i, 0)),
        pl.BlockSpec((1, C), lambda i: (0, 0)),
    ],
    out_shape=[
        jax.ShapeDtypeStruct((N, 1), jnp.float32),
        jax.ShapeDtypeStruct((N, 1), jnp.float32),
        jax.ShapeDtypeStruct((1, C), jnp.float32),
    ],
)


def _logits_body(a1_hbm, a2_hbm, src_hbm, dst_hbm, out_hbm,
                 a1_v, a2_v, sidx_v, didx_v, lg_v):
    c_id = lax.axis_index("c")
    s_id = lax.axis_index("s")
    wid = s_id * NC + c_id
    eoff = wid * EPW
    pltpu.sync_copy(a1_hbm, a1_v)
    pltpu.sync_copy(a2_hbm, a2_v)
    pltpu.sync_copy(src_hbm.at[pl.ds(eoff, EPW)], sidx_v)
    pltpu.sync_copy(dst_hbm.at[pl.ds(eoff, EPW)], didx_v)

    def body(g, carry):
        o = pl.multiple_of(g * 16, 16)
        sv = sidx_v[pl.ds(o, 16)]
        dv = didx_v[pl.ds(o, 16)]
        lg_v[pl.ds(o, 16)] = (plsc.load_gather(a1_v, [sv])
                              + plsc.load_gather(a2_v, [dv]))
        return carry

    lax.fori_loop(0, EPW // 16, body, 0)
    pltpu.sync_copy(lg_v, out_hbm.at[pl.ds(eoff, EPW)])


_logits_call = pl.kernel(
    _logits_body,
    out_type=jax.ShapeDtypeStruct((E,), jnp.float32),
    mesh=plsc.VectorSubcoreMesh(core_axis_name="c", subcore_axis_name="s"),
    scratch_types=[
        pltpu.VMEM((N,), jnp.float32),
        pltpu.VMEM((N,), jnp.float32),
        pltpu.VMEM((EPW,), jnp.int32),
        pltpu.VMEM((EPW,), jnp.int32),
        pltpu.VMEM((EPW,), jnp.float32),
    ],
    compiler_params=pltpu.CompilerParams(needs_layout_passes=False, use_tc_tiling_on_sc=False),
)


def _softmax_body(l_ref, o_ref):
    l = l_ref[...]
    m = jnp.max(l)
    e = jnp.exp(l - m)
    o_ref[...] = e / jnp.sum(e)


_softmax_call = pl.pallas_call(
    _softmax_body,
    out_shape=jax.ShapeDtypeStruct((E // C, C), jnp.float32),
)


def kernel(x, edge_index, node_type, edge_type, node_type_table,
           edge_type_table, WN_w, WN_b, WR_w, WR_b, A_w, A_b):
    src = edge_index[0].astype(jnp.int32)
    dst = edge_index[1].astype(jnp.int32)
    nt_i = node_type.astype(jnp.int32)
    et_i = edge_type.astype(jnp.int32)

    sne = _sne_call(x, nt_i.reshape(N, 1), node_type_table)

    padi = jnp.arange(EP - E, dtype=jnp.int32)
    srcp = jnp.concatenate([src, padi % N])
    dstp = jnp.concatenate([dst, N + padi % (NP - N)])
    etp = jnp.concatenate([et_i, padi % 8])
    oht = jnp.eye(8, HW, dtype=jnp.float32)
    zrow = jnp.zeros((CH, C), jnp.float32)
    zhist = jnp.zeros((CH, HW), jnp.float32)
    s_part, hist_part = _seg_call(sne, srcp, dstp, etp, oht, zrow, zhist)

    w1 = WN_w[:, :C].T
    w2 = WN_w[:, C:].T
    wrt = WR_w.T
    a12 = A_w[0].reshape(2, C).T
    a1_out, a2_out, ge = _combine_call(sne, s_part, hist_part, edge_type_table,
                                       w1, w2, wrt,
                                       WN_b.reshape(1, C), WR_b.reshape(1, C),
                                       a12)

    logits = _logits_call(a1_out.reshape(N), a2_out.reshape(N), src, dst)
    aw = _softmax_call(logits.reshape(E // C, C)).reshape(E)
    return ge.reshape(C), aw
